# Initial kernel scaffold; baseline (speedup 1.0000x reference)
import jax
import jax.numpy as jnp
from jax.experimental import pallas as pl


def kernel(hidden_states, router_w, gate_w, up_w, down_w, shared_gate_w, shared_up_w, shared_down_w):
    raise NotImplementedError("write your pallas kernel here")


# speedup vs baseline: 3.0678x; 3.0678x over previous
import jax
import jax.numpy as jnp
from jax.experimental import pallas as pl
from jax.experimental.pallas import tpu as pltpu

_E = 16
_H = 1024
_I = 512
_NG = 4
_GSZ = _E // _NG
_RSF = 2.5
_EPS = 1e-20


def _sigmoid(v):
    return 1.0 / (1.0 + jnp.exp(-v))


def _moe_body(x_ref, rwt_ref, gate_ref, up_ref, down_ref, sg_ref, su_ref, sd_ref,
              out_ref, w_ref):
    e = pl.program_id(0)
    T = x_ref.shape[0]

    @pl.when(e == 0)
    def _router():
        x = x_ref[...]
        logits = jnp.dot(x, rwt_ref[...], preferred_element_type=jnp.float32)
        scores = _sigmoid(logits)
        gsums = []
        for g in range(_NG):
            a = scores[:, 4 * g + 0:4 * g + 1]
            b = scores[:, 4 * g + 1:4 * g + 2]
            c = scores[:, 4 * g + 2:4 * g + 3]
            d = scores[:, 4 * g + 3:4 * g + 4]
            s1 = jnp.maximum(a, b); s2 = jnp.minimum(a, b)
            s3 = jnp.maximum(c, d); s4 = jnp.minimum(c, d)
            m = jnp.maximum(s1, s3)
            sec = jnp.maximum(jnp.minimum(s1, s3), jnp.maximum(s2, s4))
            gsums.append(m + sec)
        gs = jnp.concatenate(gsums, axis=1)
        cidx = jax.lax.broadcasted_iota(jnp.int32, (T, _NG), 1)
        m1 = jnp.max(gs, axis=1, keepdims=True)
        i1 = jnp.min(jnp.where(gs == m1, cidx, 9), axis=1, keepdims=True)
        e1 = cidx == i1
        gs2 = jnp.where(e1, -jnp.inf, gs)
        m2 = jnp.max(gs2, axis=1, keepdims=True)
        i2 = jnp.min(jnp.where(gs2 == m2, cidx, 9), axis=1, keepdims=True)
        gmask = jnp.logical_or(e1, cidx == i2).astype(jnp.float32)
        emask = jnp.concatenate(
            [jnp.broadcast_to(gmask[:, g:g + 1], (T, _GSZ)) for g in range(_NG)],
            axis=1)
        masked = scores * emask
        denom = jnp.sum(masked, axis=1, keepdims=True)
        w_ref[...] = masked / (denom + _EPS) * _RSF
        out_ref[...] = jnp.zeros_like(out_ref)

    x = x_ref[...]
    is_shared = e == _E
    gw = jnp.where(is_shared, sg_ref[...], gate_ref[0])
    uw = jnp.where(is_shared, su_ref[...], up_ref[0])
    dw = jnp.where(is_shared, sd_ref[...], down_ref[0])
    g = jnp.dot(x, gw, preferred_element_type=jnp.float32)
    u = jnp.dot(x, uw, preferred_element_type=jnp.float32)
    onehot = (jax.lax.broadcasted_iota(jnp.int32, (_E, 1), 0)
              == e).astype(jnp.float32)
    wcol = jnp.where(is_shared, 1.0,
                     jnp.dot(w_ref[...], onehot, preferred_element_type=jnp.float32))
    h = g * _sigmoid(g) * u * wcol
    out_ref[...] += jnp.dot(h, dw, preferred_element_type=jnp.float32)


def kernel(hidden_states, router_w, gate_w, up_w, down_w, shared_gate_w,
           shared_up_w, shared_down_w):
    B, S, Hd = hidden_states.shape
    T = B * S
    x = hidden_states.reshape(T, Hd)
    rwt = router_w.T

    out = pl.pallas_call(
        _moe_body,
        grid=(_E + 1,),
        in_specs=[
            pl.BlockSpec((T, _H), lambda e: (0, 0)),
            pl.BlockSpec((_H, _E), lambda e: (0, 0)),
            pl.BlockSpec((1, _H, _I), lambda e: (jnp.minimum(e, _E - 1), 0, 0)),
            pl.BlockSpec((1, _H, _I), lambda e: (jnp.minimum(e, _E - 1), 0, 0)),
            pl.BlockSpec((1, _I, _H), lambda e: (jnp.minimum(e, _E - 1), 0, 0)),
            pl.BlockSpec((_H, _I), lambda e: (0, 0)),
            pl.BlockSpec((_H, _I), lambda e: (0, 0)),
            pl.BlockSpec((_I, _H), lambda e: (0, 0)),
        ],
        out_specs=pl.BlockSpec((T, _H), lambda e: (0, 0)),
        out_shape=jax.ShapeDtypeStruct((T, _H), jnp.float32),
        scratch_shapes=[pltpu.VMEM((T, _E), jnp.float32)],
        compiler_params=pltpu.CompilerParams(
            dimension_semantics=("arbitrary",)),
    )(x, rwt, gate_w, up_w, down_w, shared_gate_w, shared_up_w, shared_down_w)
    return out.reshape(B, S, Hd)
